# Initial kernel scaffold; baseline (speedup 1.0000x reference)
import jax
import jax.numpy as jnp
from jax.experimental import pallas as pl


def kernel(query_token_ids, query_attention_mask, doc_token_ids, doc_attention_mask, neg_token_ids, neg_attention_mask, origin_q_emb, origin_d_emb, origin_n_emb, doc_ids, neg_ids, R, codebook):
    raise NotImplementedError("write your pallas kernel here")


# speedup vs baseline: 12.1624x; 12.1624x over previous
import functools

import jax
import jax.numpy as jnp
from jax.experimental import pallas as pl
from jax.experimental.pallas import tpu as pltpu


def _prep_body(x_ref, r_ref, cbm_ref, cbmt_ref, rot_ref, qnt_ref, *, m_sub, kc):
    x = x_ref[...]
    rot = jnp.dot(x, r_ref[...])
    rot_ref[...] = rot
    cbm = cbm_ref[...]
    n2 = jnp.sum(cbm * cbm, axis=0, keepdims=True)
    ip = jnp.dot(rot, cbm)
    dist = n2 - 2.0 * ip
    rb = dist.shape[0]
    iota = jax.lax.broadcasted_iota(jnp.int32, (rb, kc), 1)
    oh_blocks = []
    for m in range(m_sub):
        dm = dist[:, m * kc:(m + 1) * kc]
        minv = jnp.min(dm, axis=1, keepdims=True)
        cand = jnp.where(dm == minv, iota, kc)
        idx = jnp.min(cand, axis=1, keepdims=True)
        oh_blocks.append((iota == idx).astype(jnp.float32))
    oh = jnp.concatenate(oh_blocks, axis=1)
    qnt_ref[...] = jnp.dot(oh, cbmt_ref[...])


def _loss_body(oq_ref, r_ref, co_ref, cs_ref, cp_ref, od_ref, op_ref):
    oq = oq_ref[...]
    rq = jnp.dot(oq, r_ref[...])
    dims = (((1,), (1,)), ((), ()))

    t = jax.lax.dot_general(oq, co_ref[...], dims)
    t = t - jnp.max(t, axis=1, keepdims=True)
    et = jnp.exp(t)
    w = et / jnp.sum(et, axis=1, keepdims=True)

    def student_term(c_ref):
        s = jax.lax.dot_general(rq, c_ref[...], dims)
        s = s - jnp.max(s, axis=1, keepdims=True)
        es = jnp.exp(s)
        p = es / jnp.sum(es, axis=1, keepdims=True)
        return jnp.sum(w * jnp.log(p + 1e-6))

    dense_part = student_term(cs_ref)
    pq_part = student_term(cp_ref)
    od_ref[...] = jnp.full(od_ref.shape, dense_part, jnp.float32)
    op_ref[...] = jnp.full(op_ref.shape, pq_part, jnp.float32)


def kernel(query_token_ids, query_attention_mask, doc_token_ids,
           doc_attention_mask, neg_token_ids, neg_attention_mask,
           origin_q_emb, origin_d_emb, origin_n_emb, doc_ids, neg_ids,
           R, codebook):
    b, emb = origin_q_emb.shape
    m_sub, kc, d_sub = codebook.shape
    mk = m_sub * kc
    n2 = 2 * b

    eye = jnp.eye(m_sub, dtype=codebook.dtype)
    t = codebook.transpose(0, 2, 1)
    cbm = (eye[:, None, :, None] * t[:, :, None, :]).reshape(emb, mk)
    cbmt = cbm.T

    c_in = jnp.concatenate([origin_d_emb, origin_n_emb], axis=0)

    rb1 = min(256, n2)
    g1 = n2 // rb1
    rot_q = pl.pallas_call(
        functools.partial(_prep_body, m_sub=m_sub, kc=kc),
        grid=(g1,),
        in_specs=[
            pl.BlockSpec((rb1, emb), lambda i: (i, 0)),
            pl.BlockSpec((emb, emb), lambda i: (0, 0)),
            pl.BlockSpec((emb, mk), lambda i: (0, 0)),
            pl.BlockSpec((mk, emb), lambda i: (0, 0)),
        ],
        out_specs=[
            pl.BlockSpec((rb1, emb), lambda i: (i, 0)),
            pl.BlockSpec((rb1, emb), lambda i: (i, 0)),
        ],
        out_shape=[
            jax.ShapeDtypeStruct((n2, emb), jnp.float32),
            jax.ShapeDtypeStruct((n2, emb), jnp.float32),
        ],
    )(c_in, R, cbm, cbmt)
    c_s, c_p = rot_q

    rb2 = min(256, b)
    g2 = b // rb2
    partials = pl.pallas_call(
        _loss_body,
        grid=(g2,),
        in_specs=[
            pl.BlockSpec((rb2, emb), lambda i: (i, 0)),
            pl.BlockSpec((emb, emb), lambda i: (0, 0)),
            pl.BlockSpec((n2, emb), lambda i: (0, 0)),
            pl.BlockSpec((n2, emb), lambda i: (0, 0)),
            pl.BlockSpec((n2, emb), lambda i: (0, 0)),
        ],
        out_specs=[
            pl.BlockSpec((1, 8, 128), lambda i: (i, 0, 0)),
            pl.BlockSpec((1, 8, 128), lambda i: (i, 0, 0)),
        ],
        out_shape=[
            jax.ShapeDtypeStruct((g2, 8, 128), jnp.float32),
            jax.ShapeDtypeStruct((g2, 8, 128), jnp.float32),
        ],
    )(origin_q_emb, R, c_in, c_s, c_p)

    dense_loss = -jnp.sum(partials[0][:, 0, 0]) / b
    pq_loss = -jnp.sum(partials[1][:, 0, 0]) / b
    ivf_loss = jnp.asarray(0.0, dtype=jnp.float32)
    return (dense_loss, ivf_loss, pq_loss)


# speedup vs baseline: 12.1822x; 1.0016x over previous
import functools

import jax
import jax.numpy as jnp
from jax.experimental import pallas as pl
from jax.experimental.pallas import tpu as pltpu


def _prep_body(x_ref, r_ref, cbm_ref, cbmt_ref, rot_ref, qnt_ref, *, m_sub, kc):
    f32 = jnp.float32
    x16 = x_ref[...].astype(jnp.bfloat16)
    r16 = r_ref[...].astype(jnp.bfloat16)
    rot = jnp.dot(x16, r16, preferred_element_type=f32)
    rot_ref[...] = rot
    cbm = cbm_ref[...]
    n2 = jnp.sum(cbm * cbm, axis=0, keepdims=True)
    ip = jnp.dot(rot.astype(jnp.bfloat16), cbm.astype(jnp.bfloat16),
                 preferred_element_type=f32)
    dist = n2 - 2.0 * ip
    rb = dist.shape[0]
    iota = jax.lax.broadcasted_iota(jnp.int32, (rb, kc), 1)
    oh_blocks = []
    for m in range(m_sub):
        dm = dist[:, m * kc:(m + 1) * kc]
        minv = jnp.min(dm, axis=1, keepdims=True)
        cand = jnp.where(dm == minv, iota, kc)
        idx = jnp.min(cand, axis=1, keepdims=True)
        oh_blocks.append((iota == idx).astype(jnp.float32))
    oh = jnp.concatenate(oh_blocks, axis=1)
    qnt_ref[...] = jnp.dot(oh.astype(jnp.bfloat16),
                           cbmt_ref[...].astype(jnp.bfloat16),
                           preferred_element_type=f32)


def _loss_body(oq_ref, r_ref, co_ref, cs_ref, cp_ref, od_ref, op_ref):
    f32, bf16 = jnp.float32, jnp.bfloat16
    oq16 = oq_ref[...].astype(bf16)
    r16 = r_ref[...].astype(bf16)
    rq16 = jnp.dot(oq16, r16, preferred_element_type=f32).astype(bf16)
    dims = (((1,), (1,)), ((), ()))

    co16 = co_ref[...].astype(bf16)
    t = jax.lax.dot_general(oq16, co16, dims,
                            preferred_element_type=f32)
    t = t - jnp.max(t, axis=1, keepdims=True)
    et = jnp.exp(t)
    st = jnp.sum(et, axis=1, keepdims=True)

    def student_term(c_ref):
        c16 = c_ref[...].astype(bf16)
        s = jax.lax.dot_general(rq16, c16, dims, preferred_element_type=f32)
        s = s - jnp.max(s, axis=1, keepdims=True)
        es = jnp.exp(s)
        ss = jnp.sum(es, axis=1, keepdims=True)
        num = jnp.sum(et * jnp.log(es + 1e-6 * ss), axis=1, keepdims=True)
        return jnp.sum(num / st - jnp.log(ss))

    dense_part = student_term(cs_ref)
    pq_part = student_term(cp_ref)
    od_ref[...] = jnp.full(od_ref.shape, dense_part, jnp.float32)
    op_ref[...] = jnp.full(op_ref.shape, pq_part, jnp.float32)


def kernel(query_token_ids, query_attention_mask, doc_token_ids,
           doc_attention_mask, neg_token_ids, neg_attention_mask,
           origin_q_emb, origin_d_emb, origin_n_emb, doc_ids, neg_ids,
           R, codebook):
    b, emb = origin_q_emb.shape
    m_sub, kc, d_sub = codebook.shape
    mk = m_sub * kc
    n2 = 2 * b

    eye = jnp.eye(m_sub, dtype=codebook.dtype)
    t = codebook.transpose(0, 2, 1)
    cbm = (eye[:, None, :, None] * t[:, :, None, :]).reshape(emb, mk)
    cbmt = cbm.T

    c_in = jnp.concatenate([origin_d_emb, origin_n_emb], axis=0)

    rb1 = min(256, n2)
    g1 = n2 // rb1
    rot_q = pl.pallas_call(
        functools.partial(_prep_body, m_sub=m_sub, kc=kc),
        grid=(g1,),
        in_specs=[
            pl.BlockSpec((rb1, emb), lambda i: (i, 0)),
            pl.BlockSpec((emb, emb), lambda i: (0, 0)),
            pl.BlockSpec((emb, mk), lambda i: (0, 0)),
            pl.BlockSpec((mk, emb), lambda i: (0, 0)),
        ],
        out_specs=[
            pl.BlockSpec((rb1, emb), lambda i: (i, 0)),
            pl.BlockSpec((rb1, emb), lambda i: (i, 0)),
        ],
        out_shape=[
            jax.ShapeDtypeStruct((n2, emb), jnp.float32),
            jax.ShapeDtypeStruct((n2, emb), jnp.float32),
        ],
        compiler_params=pltpu.CompilerParams(
            dimension_semantics=("parallel",)),
    )(c_in, R, cbm, cbmt)
    c_s, c_p = rot_q

    rb2 = min(256, b)
    g2 = b // rb2
    partials = pl.pallas_call(
        _loss_body,
        grid=(g2,),
        in_specs=[
            pl.BlockSpec((rb2, emb), lambda i: (i, 0)),
            pl.BlockSpec((emb, emb), lambda i: (0, 0)),
            pl.BlockSpec((n2, emb), lambda i: (0, 0)),
            pl.BlockSpec((n2, emb), lambda i: (0, 0)),
            pl.BlockSpec((n2, emb), lambda i: (0, 0)),
        ],
        out_specs=[
            pl.BlockSpec((1, 8, 128), lambda i: (i, 0, 0)),
            pl.BlockSpec((1, 8, 128), lambda i: (i, 0, 0)),
        ],
        out_shape=[
            jax.ShapeDtypeStruct((g2, 8, 128), jnp.float32),
            jax.ShapeDtypeStruct((g2, 8, 128), jnp.float32),
        ],
        compiler_params=pltpu.CompilerParams(
            dimension_semantics=("parallel",)),
    )(origin_q_emb, R, c_in, c_s, c_p)

    dense_loss = -jnp.sum(partials[0][:, 0, 0]) / b
    pq_loss = -jnp.sum(partials[1][:, 0, 0]) / b
    ivf_loss = jnp.asarray(0.0, dtype=jnp.float32)
    return (dense_loss, ivf_loss, pq_loss)


# speedup vs baseline: 17.7419x; 1.4564x over previous
import functools

import jax
import jax.numpy as jnp
from jax.experimental import pallas as pl
from jax.experimental.pallas import tpu as pltpu


def _prep_body(x_ref, r_ref, cbmta_ref, cbmt_ref, rot_ref, qnt_ref, *,
               m_sub, kc):
    f32, bf16 = jnp.float32, jnp.bfloat16
    x16 = x_ref[...]
    rot = jnp.dot(x16, r_ref[...], preferred_element_type=f32)
    rot16 = rot.astype(bf16)
    rot_ref[...] = rot16
    rb = rot16.shape[0]
    rot_aug = jnp.concatenate(
        [rot16, jnp.ones((rb, 1), bf16)], axis=1)
    dims_t = (((1,), (1,)), ((), ()))
    dist_t = jax.lax.dot_general(cbmta_ref[...], rot_aug, dims_t,
                                 preferred_element_type=f32)
    d3 = dist_t.reshape(m_sub, kc, rb)
    min3 = jnp.min(d3, axis=1, keepdims=True)
    ri = jax.lax.broadcasted_iota(jnp.int32, (m_sub, kc, rb), 1)
    cand = jnp.where(d3 == min3, ri, kc)
    idx3 = jnp.min(cand, axis=1, keepdims=True)
    oh_t = (ri == idx3).astype(bf16).reshape(m_sub * kc, rb)
    dims_c = (((0,), (0,)), ((), ()))
    qnt = jax.lax.dot_general(oh_t, cbmt_ref[...], dims_c,
                              preferred_element_type=f32)
    qnt_ref[...] = qnt.astype(bf16)


def _loss_body(oq_ref, r_ref, co_ref, cs_ref, cp_ref, od_ref, op_ref):
    f32, bf16 = jnp.float32, jnp.bfloat16
    oq16 = oq_ref[...]
    rq16 = jnp.dot(oq16, r_ref[...], preferred_element_type=f32).astype(bf16)
    dims = (((1,), (1,)), ((), ()))

    t = jax.lax.dot_general(oq16, co_ref[...], dims,
                            preferred_element_type=f32)
    t = t - jnp.max(t, axis=1, keepdims=True)
    et = jnp.exp(t)
    st = jnp.sum(et, axis=1, keepdims=True)

    def student_term(c_ref):
        s = jax.lax.dot_general(rq16, c_ref[...], dims,
                                preferred_element_type=f32)
        s = s - jnp.max(s, axis=1, keepdims=True)
        es = jnp.exp(s)
        ss = jnp.sum(es, axis=1, keepdims=True)
        num = jnp.sum(et * jnp.log(es + 1e-6 * ss), axis=1, keepdims=True)
        return jnp.sum(num / st - jnp.log(ss))

    dense_part = student_term(cs_ref)
    pq_part = student_term(cp_ref)
    od_ref[...] = jnp.full(od_ref.shape, dense_part, jnp.float32)
    op_ref[...] = jnp.full(op_ref.shape, pq_part, jnp.float32)


def kernel(query_token_ids, query_attention_mask, doc_token_ids,
           doc_attention_mask, neg_token_ids, neg_attention_mask,
           origin_q_emb, origin_d_emb, origin_n_emb, doc_ids, neg_ids,
           R, codebook):
    f32, bf16 = jnp.float32, jnp.bfloat16
    b, emb = origin_q_emb.shape
    m_sub, kc, d_sub = codebook.shape
    mk = m_sub * kc
    n2 = 2 * b

    eye = jnp.eye(m_sub, dtype=codebook.dtype)
    cbmt = (eye[:, :, None, None] * codebook[:, None, :, :]) \
        .transpose(0, 2, 1, 3).reshape(mk, emb)
    n2col = jnp.sum(codebook * codebook, axis=-1).reshape(mk, 1)
    cbmta = jnp.concatenate([-2.0 * cbmt, n2col], axis=1)

    c_in = jnp.concatenate([origin_d_emb, origin_n_emb], axis=0)
    c_in16 = c_in.astype(bf16)
    oq16 = origin_q_emb.astype(bf16)
    r16 = R.astype(bf16)
    cbmt16 = cbmt.astype(bf16)
    cbmta16 = cbmta.astype(bf16)

    rb1 = min(256, n2)
    g1 = n2 // rb1
    c_s, c_p = pl.pallas_call(
        functools.partial(_prep_body, m_sub=m_sub, kc=kc),
        grid=(g1,),
        in_specs=[
            pl.BlockSpec((rb1, emb), lambda i: (i, 0)),
            pl.BlockSpec((emb, emb), lambda i: (0, 0)),
            pl.BlockSpec((mk, emb + 1), lambda i: (0, 0)),
            pl.BlockSpec((mk, emb), lambda i: (0, 0)),
        ],
        out_specs=[
            pl.BlockSpec((rb1, emb), lambda i: (i, 0)),
            pl.BlockSpec((rb1, emb), lambda i: (i, 0)),
        ],
        out_shape=[
            jax.ShapeDtypeStruct((n2, emb), bf16),
            jax.ShapeDtypeStruct((n2, emb), bf16),
        ],
        compiler_params=pltpu.CompilerParams(
            dimension_semantics=("parallel",)),
    )(c_in16, r16, cbmta16, cbmt16)

    rb2 = min(256, b)
    g2 = b // rb2
    partials = pl.pallas_call(
        _loss_body,
        grid=(g2,),
        in_specs=[
            pl.BlockSpec((rb2, emb), lambda i: (i, 0)),
            pl.BlockSpec((emb, emb), lambda i: (0, 0)),
            pl.BlockSpec((n2, emb), lambda i: (0, 0)),
            pl.BlockSpec((n2, emb), lambda i: (0, 0)),
            pl.BlockSpec((n2, emb), lambda i: (0, 0)),
        ],
        out_specs=[
            pl.BlockSpec((1, 8, 128), lambda i: (i, 0, 0)),
            pl.BlockSpec((1, 8, 128), lambda i: (i, 0, 0)),
        ],
        out_shape=[
            jax.ShapeDtypeStruct((g2, 8, 128), jnp.float32),
            jax.ShapeDtypeStruct((g2, 8, 128), jnp.float32),
        ],
        compiler_params=pltpu.CompilerParams(
            dimension_semantics=("parallel",)),
    )(oq16, r16, c_in16, c_s, c_p)

    dense_loss = -jnp.sum(partials[0][:, 0, 0]) / b
    pq_loss = -jnp.sum(partials[1][:, 0, 0]) / b
    ivf_loss = jnp.asarray(0.0, dtype=f32)
    return (dense_loss, ivf_loss, pq_loss)


# speedup vs baseline: 18.4653x; 1.0408x over previous
import functools

import jax
import jax.numpy as jnp
from jax.experimental import pallas as pl
from jax.experimental.pallas import tpu as pltpu


def _body(oq_ref, cin_ref, r_ref, cbmta_ref, cbmt_ref, od_ref, op_ref,
          cs_ref, cp_ref, *, m_sub, kc, rb1, g1, rb2, g2):
    f32, bf16 = jnp.float32, jnp.bfloat16
    i = pl.program_id(0)

    @pl.when(i < g1)
    def _prep():
        x16 = cin_ref[pl.ds(i * rb1, rb1), :]
        rot = jnp.dot(x16, r_ref[...], preferred_element_type=f32)
        rot16 = rot.astype(bf16)
        cs_ref[pl.ds(i * rb1, rb1), :] = rot16
        rot_aug = jnp.concatenate(
            [rot16, jnp.ones((rb1, 1), bf16)], axis=1)
        dims_t = (((1,), (1,)), ((), ()))
        dist_t = jax.lax.dot_general(cbmta_ref[...], rot_aug, dims_t,
                                     preferred_element_type=f32)
        d3 = dist_t.reshape(m_sub, kc, rb1)
        min3 = jnp.min(d3, axis=1, keepdims=True)
        oh_t = (d3 == min3).astype(bf16).reshape(m_sub * kc, rb1)
        dims_c = (((0,), (0,)), ((), ()))
        qnt = jax.lax.dot_general(oh_t, cbmt_ref[...], dims_c,
                                  preferred_element_type=f32)
        cp_ref[pl.ds(i * rb1, rb1), :] = qnt.astype(bf16)

    @pl.when(i >= g1)
    def _loss():
        li = i - g1
        oq16 = oq_ref[pl.ds(li * rb2, rb2), :]
        rq16 = jnp.dot(oq16, r_ref[...],
                       preferred_element_type=f32).astype(bf16)
        dims = (((1,), (1,)), ((), ()))

        t = jax.lax.dot_general(oq16, cin_ref[...], dims,
                                preferred_element_type=f32)
        t = t - jnp.max(t, axis=1, keepdims=True)
        et = jnp.exp(t)
        st = jnp.sum(et, axis=1, keepdims=True)

        def student_term(c_ref):
            s = jax.lax.dot_general(rq16, c_ref[...], dims,
                                    preferred_element_type=f32)
            s = s - jnp.max(s, axis=1, keepdims=True)
            es = jnp.exp(s)
            ss = jnp.sum(es, axis=1, keepdims=True)
            num = jnp.sum(et * jnp.log(es + 1e-6 * ss), axis=1, keepdims=True)
            return jnp.sum(num / st - jnp.log(ss))

        dense_part = student_term(cs_ref)
        pq_part = student_term(cp_ref)
        od_ref[pl.ds(li, 1), :, :] = jnp.full((1, 8, 128), dense_part, f32)
        op_ref[pl.ds(li, 1), :, :] = jnp.full((1, 8, 128), pq_part, f32)


def kernel(query_token_ids, query_attention_mask, doc_token_ids,
           doc_attention_mask, neg_token_ids, neg_attention_mask,
           origin_q_emb, origin_d_emb, origin_n_emb, doc_ids, neg_ids,
           R, codebook):
    f32, bf16 = jnp.float32, jnp.bfloat16
    b, emb = origin_q_emb.shape
    m_sub, kc, d_sub = codebook.shape
    mk = m_sub * kc
    n2 = 2 * b

    eye = jnp.eye(m_sub, dtype=codebook.dtype)
    cbmt = (eye[:, :, None, None] * codebook[:, None, :, :]) \
        .transpose(0, 2, 1, 3).reshape(mk, emb)
    n2col = jnp.sum(codebook * codebook, axis=-1).reshape(mk, 1)
    cbmta = jnp.concatenate([-2.0 * cbmt, n2col], axis=1)

    c_in16 = jnp.concatenate([origin_d_emb, origin_n_emb],
                             axis=0).astype(bf16)
    oq16 = origin_q_emb.astype(bf16)
    r16 = R.astype(bf16)
    cbmt16 = cbmt.astype(bf16)
    cbmta16 = cbmta.astype(bf16)

    rb1 = min(256, n2)
    g1 = n2 // rb1
    rb2 = min(256, b)
    g2 = b // rb2

    full = lambda shape: pl.BlockSpec(shape, lambda i: tuple(0 for _ in shape))
    partials = pl.pallas_call(
        functools.partial(_body, m_sub=m_sub, kc=kc,
                          rb1=rb1, g1=g1, rb2=rb2, g2=g2),
        grid=(g1 + g2,),
        in_specs=[
            full((b, emb)),
            full((n2, emb)),
            full((emb, emb)),
            full((mk, emb + 1)),
            full((mk, emb)),
        ],
        out_specs=[
            full((g2, 8, 128)),
            full((g2, 8, 128)),
        ],
        out_shape=[
            jax.ShapeDtypeStruct((g2, 8, 128), f32),
            jax.ShapeDtypeStruct((g2, 8, 128), f32),
        ],
        scratch_shapes=[
            pltpu.VMEM((n2, emb), bf16),
            pltpu.VMEM((n2, emb), bf16),
        ],
        compiler_params=pltpu.CompilerParams(
            dimension_semantics=("arbitrary",)),
    )(oq16, c_in16, r16, cbmta16, cbmt16)

    dense_loss = -jnp.sum(partials[0][:, 0, 0]) / b
    pq_loss = -jnp.sum(partials[1][:, 0, 0]) / b
    ivf_loss = jnp.asarray(0.0, dtype=f32)
    return (dense_loss, ivf_loss, pq_loss)
